# baseline (speedup 1.0000x reference)
import jax
import jax.numpy as jnp
from jax.experimental import pallas as pl

EMBED = 2048
EXPERTS = 16
ROW_TILE = 512


def _gating_tile(x_ref, w_ref, b_ref, o_ref):
    x = x_ref[...]
    w = w_ref[...]
    scores = jax.lax.dot_general(
        x, w, (((1,), (1,)), ((), ())), preferred_element_type=jnp.float32
    )
    scores = scores + b_ref[...]
    m = jnp.max(scores, axis=1, keepdims=True)
    e = jnp.exp(scores - m)
    o_ref[...] = e / jnp.sum(e, axis=1, keepdims=True)


def kernel(x, W, b):
    target_length, batch_size, embed_dim = x.shape
    rows = target_length * batch_size
    x2 = x.reshape(rows, embed_dim)
    b2 = b.reshape(1, EXPERTS)
    grid = rows // ROW_TILE
    out = pl.pallas_call(
        _gating_tile,
        grid=(grid,),
        in_specs=[
            pl.BlockSpec((ROW_TILE, embed_dim), lambda i: (i, 0)),
            pl.BlockSpec((EXPERTS, embed_dim), lambda i: (0, 0)),
            pl.BlockSpec((1, EXPERTS), lambda i: (0, 0)),
        ],
        out_specs=pl.BlockSpec((ROW_TILE, EXPERTS), lambda i: (i, 0)),
        out_shape=jax.ShapeDtypeStruct((rows, EXPERTS), jnp.float32),
    )(x2, W, b2)
    return out.reshape(target_length, batch_size, EXPERTS)


# speedup vs baseline: 1.0557x; 1.0557x over previous
import jax
import jax.numpy as jnp
from jax.experimental import pallas as pl

EMBED = 2048
EXPERTS = 16
ROW_TILE = 1024


def _gating_tile(x_ref, w_ref, b_ref, o_ref):
    x = x_ref[...]
    w = w_ref[...]
    scores = jax.lax.dot_general(
        x, w, (((1,), (1,)), ((), ())), preferred_element_type=jnp.float32
    )
    scores = scores + b_ref[...]
    m = jnp.max(scores, axis=1, keepdims=True)
    e = jnp.exp(scores - m)
    o_ref[...] = e / jnp.sum(e, axis=1, keepdims=True)


def kernel(x, W, b):
    target_length, batch_size, embed_dim = x.shape
    rows = target_length * batch_size
    x2 = x.reshape(rows, embed_dim)
    b2 = b.reshape(1, EXPERTS)
    grid = rows // ROW_TILE
    out = pl.pallas_call(
        _gating_tile,
        grid=(grid,),
        in_specs=[
            pl.BlockSpec((ROW_TILE, embed_dim), lambda i: (i, 0)),
            pl.BlockSpec((EXPERTS, embed_dim), lambda i: (0, 0)),
            pl.BlockSpec((1, EXPERTS), lambda i: (0, 0)),
        ],
        out_specs=pl.BlockSpec((ROW_TILE, EXPERTS), lambda i: (i, 0)),
        out_shape=jax.ShapeDtypeStruct((rows, EXPERTS), jnp.float32),
    )(x2, W, b2)
    return out.reshape(target_length, batch_size, EXPERTS)
